# baseline (speedup 1.0000x reference)
import functools

import jax
import jax.numpy as jnp
from jax import lax
from jax.experimental import pallas as pl
from jax.experimental.pallas import tpu as pltpu
from jax.experimental.pallas import tpu_sc as plsc

N = 10000
N_PAD = 10240
E = 320000
D_IN = 128
D_HID = 128
N_CLS = 40
D_CLS_PAD = 48

NC = 2
NS = 16
ROWS_PER_SUB = N_PAD // NS
E_PER_CORE = E // NC
E_PER_TILE = E_PER_CORE // NS
CHUNK = 128
TILE_CHUNKS = 80
NGRP = 2
GRP = TILE_CHUNKS // NGRP
E_PAD = NC * NS * CHUNK * TILE_CHUNKS
PAD_IDX = N_PAD - 1
IDX_ROWS = E_PAD // CHUNK

_R = 1000
_GRID = N // _R


def _sc_mesh():
  return plsc.VectorSubcoreMesh(core_axis_name="c", subcore_axis_name="s")


def _deg_body(src2_hbm, dst2_hbm, zeros_hbm, ones_hbm, out_hbm,
              isrc_v, idst_v, ones_v, degd_sh, degs_sh):
  c = lax.axis_index("c")
  s = lax.axis_index("s")
  r0 = s * ROWS_PER_SUB
  rows = pl.ds(r0, ROWS_PER_SUB)
  pltpu.sync_copy(zeros_hbm, degd_sh.at[rows])
  pltpu.sync_copy(zeros_hbm, degs_sh.at[rows])
  pltpu.sync_copy(ones_hbm, ones_v)
  row0 = (c * NS + s) * TILE_CHUNKS
  pltpu.sync_copy(src2_hbm.at[pl.ds(row0, TILE_CHUNKS)], isrc_v)
  pltpu.sync_copy(dst2_hbm.at[pl.ds(row0, TILE_CHUNKS)], idst_v)
  plsc.subcore_barrier()

  def step(i, carry):
    pltpu.sync_copy(ones_v, degd_sh.at[idst_v.at[i]], add=True)
    pltpu.sync_copy(ones_v, degs_sh.at[isrc_v.at[i]], add=True)
    return carry

  lax.fori_loop(0, TILE_CHUNKS, step, 0)
  plsc.subcore_barrier()
  pltpu.sync_copy(degd_sh.at[rows], out_hbm.at[c, 0, rows])
  pltpu.sync_copy(degs_sh.at[rows], out_hbm.at[c, 1, rows])


def _deg_call(src2, dst2):
  zeros = jnp.zeros((ROWS_PER_SUB, 16), jnp.float32)
  ones = jnp.ones((CHUNK, 16), jnp.float32)
  fn = pl.kernel(
      _deg_body,
      out_type=jax.ShapeDtypeStruct((NC, 2, N_PAD, 16), jnp.float32),
      mesh=_sc_mesh(),
      compiler_params=pltpu.CompilerParams(use_tc_tiling_on_sc=False),
      scratch_types=[
          pltpu.VMEM((TILE_CHUNKS, CHUNK), jnp.int32),
          pltpu.VMEM((TILE_CHUNKS, CHUNK), jnp.int32),
          pltpu.VMEM((CHUNK, 16), jnp.float32),
          pltpu.VMEM_SHARED((N_PAD, 16), jnp.float32),
          pltpu.VMEM_SHARED((N_PAD, 16), jnp.float32),
      ],
  )
  return fn(src2, dst2, zeros, ones)


def _agg_body(h_hbm, src2_hbm, dst2_hbm, zeros_hbm, out_hbm,
              isrc_v, idst_v, rows_a, rows_b, acc_sh, sem_a, sem_b):
  c = lax.axis_index("c")
  s = lax.axis_index("s")
  r0 = s * ROWS_PER_SUB
  rows = pl.ds(r0, ROWS_PER_SUB)
  pltpu.sync_copy(zeros_hbm, acc_sh.at[rows])
  tile_row0 = (c * NS + s) * TILE_CHUNKS
  plsc.subcore_barrier()

  for g in range(NGRP):
    row0 = tile_row0 + g * GRP
    pltpu.sync_copy(src2_hbm.at[pl.ds(row0, GRP)], isrc_v)
    pltpu.sync_copy(dst2_hbm.at[pl.ds(row0, GRP)], idst_v)
    pltpu.async_copy(h_hbm.at[isrc_v.at[0]], rows_a, sem_a)
    pltpu.async_copy(h_hbm.at[isrc_v.at[1]], rows_b, sem_b)

    def step(i, carry):
      ja = 2 * i
      jb = ja + 1
      pltpu.make_async_copy(h_hbm.at[isrc_v.at[ja]], rows_a, sem_a).wait()
      pltpu.sync_copy(rows_a, acc_sh.at[idst_v.at[ja]], add=True)
      pltpu.async_copy(h_hbm.at[isrc_v.at[ja + 2]], rows_a, sem_a)
      pltpu.make_async_copy(h_hbm.at[isrc_v.at[jb]], rows_b, sem_b).wait()
      pltpu.sync_copy(rows_b, acc_sh.at[idst_v.at[jb]], add=True)
      pltpu.async_copy(h_hbm.at[isrc_v.at[jb + 2]], rows_b, sem_b)
      return carry

    lax.fori_loop(0, GRP // 2 - 1, step, 0)
    last = GRP - 2
    pltpu.make_async_copy(h_hbm.at[isrc_v.at[last]], rows_a, sem_a).wait()
    pltpu.sync_copy(rows_a, acc_sh.at[idst_v.at[last]], add=True)
    pltpu.make_async_copy(h_hbm.at[isrc_v.at[last + 1]], rows_b, sem_b).wait()
    pltpu.sync_copy(rows_b, acc_sh.at[idst_v.at[last + 1]], add=True)

  plsc.subcore_barrier()
  pltpu.sync_copy(acc_sh.at[rows], out_hbm.at[c, rows])


def _make_agg(d):
  params = None
  if d % 128 != 0:
    params = pltpu.CompilerParams(use_tc_tiling_on_sc=False)
  return pl.kernel(
      _agg_body,
      out_type=jax.ShapeDtypeStruct((NC, N_PAD, d), jnp.float32),
      mesh=_sc_mesh(),
      compiler_params=params,
      scratch_types=[
          pltpu.VMEM((GRP, CHUNK), jnp.int32),
          pltpu.VMEM((GRP, CHUNK), jnp.int32),
          pltpu.VMEM((CHUNK, d), jnp.float32),
          pltpu.VMEM((CHUNK, d), jnp.float32),
          pltpu.VMEM_SHARED((N_PAD, d), jnp.float32),
          pltpu.SemaphoreType.DMA,
          pltpu.SemaphoreType.DMA,
      ],
  )


def _agg(h, src2, dst2):
  d = h.shape[1]
  zeros = jnp.zeros((ROWS_PER_SUB, d), jnp.float32)
  return _make_agg(d)(h, src2, dst2, zeros)


def _pre_body(degp_ref, x_ref, hs_ref, nd_ref, ns_ref):
  dd = degp_ref[0, 0] + degp_ref[1, 0]
  dsr = degp_ref[0, 1] + degp_ref[1, 1]
  nd = lax.rsqrt(jnp.where(dd > 0.0, dd, 1.0))
  ns = lax.rsqrt(jnp.where(dsr > 0.0, dsr, 1.0))
  nd_ref[...] = nd
  ns_ref[...] = ns
  hs_ref[...] = x_ref[...] * ns[:, 0:1]


def _pre_call(degp, x):
  return pl.pallas_call(
      _pre_body,
      grid=(_GRID,),
      in_specs=[
          pl.BlockSpec((NC, 2, _R, 16), lambda i: (0, 0, i, 0)),
          pl.BlockSpec((_R, D_IN), lambda i: (i, 0)),
      ],
      out_specs=[
          pl.BlockSpec((_R, D_IN), lambda i: (i, 0)),
          pl.BlockSpec((_R, 16), lambda i: (i, 0)),
          pl.BlockSpec((_R, 16), lambda i: (i, 0)),
      ],
      out_shape=[
          jax.ShapeDtypeStruct((N_PAD, D_IN), jnp.float32),
          jax.ShapeDtypeStruct((N, 16), jnp.float32),
          jax.ShapeDtypeStruct((N, 16), jnp.float32),
      ],
  )(degp, x)


def _mid_body(p_ref, nd_ref, ns_ref, w_ref, b_ref, o_ref):
  agg = (p_ref[0] + p_ref[1]) * nd_ref[:, 0:1]
  z = jnp.dot(agg, w_ref[...], preferred_element_type=jnp.float32) + b_ref[...]
  o_ref[...] = jnp.maximum(z, 0.0) * ns_ref[:, 0:1]


def _mid_call(p, nd, ns, w, b):
  return pl.pallas_call(
      _mid_body,
      grid=(_GRID,),
      in_specs=[
          pl.BlockSpec((NC, _R, D_HID), lambda i: (0, i, 0)),
          pl.BlockSpec((_R, 16), lambda i: (i, 0)),
          pl.BlockSpec((_R, 16), lambda i: (i, 0)),
          pl.BlockSpec((D_HID, D_HID), lambda i: (0, 0)),
          pl.BlockSpec((1, D_HID), lambda i: (0, 0)),
      ],
      out_specs=pl.BlockSpec((_R, D_HID), lambda i: (i, 0)),
      out_shape=jax.ShapeDtypeStruct((N_PAD, D_HID), jnp.float32),
  )(p, nd, ns, w, b.reshape(1, -1))


def _mid3_body(p_ref, nd_ref, ns_ref, w_ref, b_ref, w3_ref, o_ref):
  agg = (p_ref[0] + p_ref[1]) * nd_ref[:, 0:1]
  z = jnp.dot(agg, w_ref[...], preferred_element_type=jnp.float32) + b_ref[...]
  h = jnp.maximum(z, 0.0) * ns_ref[:, 0:1]
  o_ref[...] = jnp.dot(h, w3_ref[...], preferred_element_type=jnp.float32)


def _mid3_call(p, nd, ns, w, b, w3p):
  return pl.pallas_call(
      _mid3_body,
      grid=(_GRID,),
      in_specs=[
          pl.BlockSpec((NC, _R, D_HID), lambda i: (0, i, 0)),
          pl.BlockSpec((_R, 16), lambda i: (i, 0)),
          pl.BlockSpec((_R, 16), lambda i: (i, 0)),
          pl.BlockSpec((D_HID, D_HID), lambda i: (0, 0)),
          pl.BlockSpec((1, D_HID), lambda i: (0, 0)),
          pl.BlockSpec((D_HID, D_CLS_PAD), lambda i: (0, 0)),
      ],
      out_specs=pl.BlockSpec((_R, D_CLS_PAD), lambda i: (i, 0)),
      out_shape=jax.ShapeDtypeStruct((N_PAD, D_CLS_PAD), jnp.float32),
  )(p, nd, ns, w, b.reshape(1, -1), w3p)


def _fin_body(p_ref, nd_ref, b_ref, o_ref):
  agg = (p_ref[0] + p_ref[1]) * nd_ref[:, 0:1]
  o_ref[...] = (agg + b_ref[...])[:, :N_CLS]


def _fin_call(p, nd, b3p):
  return pl.pallas_call(
      _fin_body,
      grid=(_GRID,),
      in_specs=[
          pl.BlockSpec((NC, _R, D_CLS_PAD), lambda i: (0, i, 0)),
          pl.BlockSpec((_R, 16), lambda i: (i, 0)),
          pl.BlockSpec((1, D_CLS_PAD), lambda i: (0, 0)),
      ],
      out_specs=pl.BlockSpec((_R, N_CLS), lambda i: (i, 0)),
      out_shape=jax.ShapeDtypeStruct((N, N_CLS), jnp.float32),
  )(p, nd, b3p)


def kernel(x, edge_index, W1, b1, W2, b2, W3, b3):
  pad = jnp.full((E_PAD - E,), PAD_IDX, jnp.int32)
  src2 = jnp.concatenate([edge_index[0], pad]).reshape(IDX_ROWS, CHUNK)
  dst2 = jnp.concatenate([edge_index[1], pad]).reshape(IDX_ROWS, CHUNK)
  w3p = jnp.pad(W3, ((0, 0), (0, D_CLS_PAD - N_CLS)))
  b3p = jnp.pad(b3, (0, D_CLS_PAD - N_CLS)).reshape(1, -1)

  degp = _deg_call(src2, dst2)
  hs, nd, ns = _pre_call(degp, x)
  p1 = _agg(hs, src2, dst2)
  h2s = _mid_call(p1, nd, ns, W1, b1)
  p2 = _agg(h2s, src2, dst2)
  m3 = _mid3_call(p2, nd, ns, W2, b2, w3p)
  p3 = _agg(m3, src2, dst2)
  return _fin_call(p3, nd, b3p)


# speedup vs baseline: 1.1946x; 1.1946x over previous
import functools

import jax
import jax.numpy as jnp
from jax import lax
from jax.experimental import pallas as pl
from jax.experimental.pallas import tpu as pltpu
from jax.experimental.pallas import tpu_sc as plsc

N = 10000
N_PAD = 10240
E = 320000
D_IN = 128
D_HID = 128
N_CLS = 40
D_CLS_PAD = 48

NC = 2
NS = 16
ROWS_PER_SUB = N_PAD // NS
E_PER_CORE = E // NC
E_PER_TILE = E_PER_CORE // NS
CHUNK = 96
TILE_CHUNKS = 106
NGRP = 1
GRP = TILE_CHUNKS // NGRP
E_PAD = NC * NS * CHUNK * TILE_CHUNKS
PAD_IDX = N_PAD - 1
IDX_ROWS = E_PAD // CHUNK

_R = 1000
_GRID = N // _R


def _sc_mesh():
  return plsc.VectorSubcoreMesh(core_axis_name="c", subcore_axis_name="s")


def _deg_body(src2_hbm, dst2_hbm, zeros_hbm, ones_hbm, out_hbm,
              isrc_v, idst_v, ones_v, degd_sh, degs_sh):
  c = lax.axis_index("c")
  s = lax.axis_index("s")
  r0 = s * ROWS_PER_SUB
  rows = pl.ds(r0, ROWS_PER_SUB)
  pltpu.sync_copy(zeros_hbm, degd_sh.at[rows])
  pltpu.sync_copy(zeros_hbm, degs_sh.at[rows])
  pltpu.sync_copy(ones_hbm, ones_v)
  row0 = (c * NS + s) * TILE_CHUNKS
  pltpu.sync_copy(src2_hbm.at[pl.ds(row0, TILE_CHUNKS)], isrc_v)
  pltpu.sync_copy(dst2_hbm.at[pl.ds(row0, TILE_CHUNKS)], idst_v)
  plsc.subcore_barrier()

  def step(i, carry):
    pltpu.sync_copy(ones_v, degd_sh.at[idst_v.at[i]], add=True)
    pltpu.sync_copy(ones_v, degs_sh.at[isrc_v.at[i]], add=True)
    return carry

  lax.fori_loop(0, TILE_CHUNKS, step, 0)
  plsc.subcore_barrier()
  pltpu.sync_copy(degd_sh.at[rows], out_hbm.at[c, 0, rows])
  pltpu.sync_copy(degs_sh.at[rows], out_hbm.at[c, 1, rows])


def _deg_call(src2, dst2):
  zeros = jnp.zeros((ROWS_PER_SUB, 16), jnp.float32)
  ones = jnp.ones((CHUNK, 16), jnp.float32)
  fn = pl.kernel(
      _deg_body,
      out_type=jax.ShapeDtypeStruct((NC, 2, N_PAD, 16), jnp.float32),
      mesh=_sc_mesh(),
      compiler_params=pltpu.CompilerParams(use_tc_tiling_on_sc=False),
      scratch_types=[
          pltpu.VMEM((TILE_CHUNKS, CHUNK), jnp.int32),
          pltpu.VMEM((TILE_CHUNKS, CHUNK), jnp.int32),
          pltpu.VMEM((CHUNK, 16), jnp.float32),
          pltpu.VMEM_SHARED((N_PAD, 16), jnp.float32),
          pltpu.VMEM_SHARED((N_PAD, 16), jnp.float32),
      ],
  )
  return fn(src2, dst2, zeros, ones)


def _agg_body(h_hbm, src2_hbm, dst2_hbm, zeros_hbm, out_hbm,
              isrc_v, idst_v, rows_a, rows_b, acc_sh, sem_a, sem_b):
  c = lax.axis_index("c")
  s = lax.axis_index("s")
  r0 = s * ROWS_PER_SUB
  rows = pl.ds(r0, ROWS_PER_SUB)
  with jax.named_scope("agg_init"):
    pltpu.sync_copy(zeros_hbm, acc_sh.at[rows])
    tile_row0 = (c * NS + s) * TILE_CHUNKS
    plsc.subcore_barrier()

  for g in range(NGRP):
    row0 = tile_row0 + g * GRP
    pltpu.sync_copy(src2_hbm.at[pl.ds(row0, GRP)], isrc_v)
    pltpu.sync_copy(dst2_hbm.at[pl.ds(row0, GRP)], idst_v)
    pltpu.async_copy(h_hbm.at[isrc_v.at[0]], rows_a, sem_a)
    pltpu.async_copy(h_hbm.at[isrc_v.at[1]], rows_b, sem_b)

    def step(i, carry):
      ja = 2 * i
      jb = ja + 1
      pltpu.make_async_copy(h_hbm.at[isrc_v.at[ja]], rows_a, sem_a).wait()
      pltpu.sync_copy(rows_a, acc_sh.at[idst_v.at[ja]], add=True)
      pltpu.async_copy(h_hbm.at[isrc_v.at[ja + 2]], rows_a, sem_a)
      pltpu.make_async_copy(h_hbm.at[isrc_v.at[jb]], rows_b, sem_b).wait()
      pltpu.sync_copy(rows_b, acc_sh.at[idst_v.at[jb]], add=True)
      pltpu.async_copy(h_hbm.at[isrc_v.at[jb + 2]], rows_b, sem_b)
      return carry

    with jax.named_scope("agg_loop"):
      lax.fori_loop(0, GRP // 2 - 1, step, 0)
    last = GRP - 2
    pltpu.make_async_copy(h_hbm.at[isrc_v.at[last]], rows_a, sem_a).wait()
    pltpu.sync_copy(rows_a, acc_sh.at[idst_v.at[last]], add=True)
    pltpu.make_async_copy(h_hbm.at[isrc_v.at[last + 1]], rows_b, sem_b).wait()
    pltpu.sync_copy(rows_b, acc_sh.at[idst_v.at[last + 1]], add=True)

  with jax.named_scope("agg_wb"):
    plsc.subcore_barrier()
    pltpu.sync_copy(acc_sh.at[rows], out_hbm.at[c, rows])


def _make_agg(d):
  params = pltpu.CompilerParams(use_tc_tiling_on_sc=False)
  return pl.kernel(
      _agg_body,
      out_type=jax.ShapeDtypeStruct((NC, N_PAD, d), jnp.float32),
      mesh=_sc_mesh(),
      compiler_params=params,
      scratch_types=[
          pltpu.VMEM((GRP, CHUNK), jnp.int32),
          pltpu.VMEM((GRP, CHUNK), jnp.int32),
          pltpu.VMEM((CHUNK, d), jnp.float32),
          pltpu.VMEM((CHUNK, d), jnp.float32),
          pltpu.VMEM_SHARED((N_PAD, d), jnp.float32),
          pltpu.SemaphoreType.DMA,
          pltpu.SemaphoreType.DMA,
      ],
  )


def _agg(h, src2, dst2):
  d = h.shape[1]
  zeros = jnp.zeros((ROWS_PER_SUB, d), jnp.float32)
  return _make_agg(d)(h, src2, dst2, zeros)


def _pre_body(degp_ref, x_ref, hs_ref, nd_ref, ns_ref):
  dd = degp_ref[0, 0] + degp_ref[1, 0]
  dsr = degp_ref[0, 1] + degp_ref[1, 1]
  nd = lax.rsqrt(jnp.where(dd > 0.0, dd, 1.0))
  ns = lax.rsqrt(jnp.where(dsr > 0.0, dsr, 1.0))
  nd_ref[...] = nd
  ns_ref[...] = ns
  hs_ref[...] = x_ref[...] * ns[:, 0:1]


def _pre_call(degp, x):
  return pl.pallas_call(
      _pre_body,
      grid=(_GRID,),
      in_specs=[
          pl.BlockSpec((NC, 2, _R, 16), lambda i: (0, 0, i, 0)),
          pl.BlockSpec((_R, D_IN), lambda i: (i, 0)),
      ],
      out_specs=[
          pl.BlockSpec((_R, D_IN), lambda i: (i, 0)),
          pl.BlockSpec((_R, 16), lambda i: (i, 0)),
          pl.BlockSpec((_R, 16), lambda i: (i, 0)),
      ],
      out_shape=[
          jax.ShapeDtypeStruct((N_PAD, D_IN), jnp.float32),
          jax.ShapeDtypeStruct((N, 16), jnp.float32),
          jax.ShapeDtypeStruct((N, 16), jnp.float32),
      ],
  )(degp, x)


def _mid_body(p_ref, nd_ref, ns_ref, w_ref, b_ref, o_ref):
  agg = (p_ref[0] + p_ref[1]) * nd_ref[:, 0:1]
  z = jnp.dot(agg, w_ref[...], preferred_element_type=jnp.float32) + b_ref[...]
  o_ref[...] = jnp.maximum(z, 0.0) * ns_ref[:, 0:1]


def _mid_call(p, nd, ns, w, b):
  return pl.pallas_call(
      _mid_body,
      grid=(_GRID,),
      in_specs=[
          pl.BlockSpec((NC, _R, D_HID), lambda i: (0, i, 0)),
          pl.BlockSpec((_R, 16), lambda i: (i, 0)),
          pl.BlockSpec((_R, 16), lambda i: (i, 0)),
          pl.BlockSpec((D_HID, D_HID), lambda i: (0, 0)),
          pl.BlockSpec((1, D_HID), lambda i: (0, 0)),
      ],
      out_specs=pl.BlockSpec((_R, D_HID), lambda i: (i, 0)),
      out_shape=jax.ShapeDtypeStruct((N_PAD, D_HID), jnp.float32),
  )(p, nd, ns, w, b.reshape(1, -1))


def _mid3_body(p_ref, nd_ref, ns_ref, w_ref, b_ref, w3_ref, o_ref):
  agg = (p_ref[0] + p_ref[1]) * nd_ref[:, 0:1]
  z = jnp.dot(agg, w_ref[...], preferred_element_type=jnp.float32) + b_ref[...]
  h = jnp.maximum(z, 0.0) * ns_ref[:, 0:1]
  o_ref[...] = jnp.dot(h, w3_ref[...], preferred_element_type=jnp.float32)


def _mid3_call(p, nd, ns, w, b, w3p):
  return pl.pallas_call(
      _mid3_body,
      grid=(_GRID,),
      in_specs=[
          pl.BlockSpec((NC, _R, D_HID), lambda i: (0, i, 0)),
          pl.BlockSpec((_R, 16), lambda i: (i, 0)),
          pl.BlockSpec((_R, 16), lambda i: (i, 0)),
          pl.BlockSpec((D_HID, D_HID), lambda i: (0, 0)),
          pl.BlockSpec((1, D_HID), lambda i: (0, 0)),
          pl.BlockSpec((D_HID, D_CLS_PAD), lambda i: (0, 0)),
      ],
      out_specs=pl.BlockSpec((_R, D_CLS_PAD), lambda i: (i, 0)),
      out_shape=jax.ShapeDtypeStruct((N_PAD, D_CLS_PAD), jnp.float32),
  )(p, nd, ns, w, b.reshape(1, -1), w3p)


def _fin_body(p_ref, nd_ref, b_ref, o_ref):
  agg = (p_ref[0] + p_ref[1]) * nd_ref[:, 0:1]
  o_ref[...] = (agg + b_ref[...])[:, :N_CLS]


def _fin_call(p, nd, b3p):
  return pl.pallas_call(
      _fin_body,
      grid=(_GRID,),
      in_specs=[
          pl.BlockSpec((NC, _R, D_CLS_PAD), lambda i: (0, i, 0)),
          pl.BlockSpec((_R, 16), lambda i: (i, 0)),
          pl.BlockSpec((1, D_CLS_PAD), lambda i: (0, 0)),
      ],
      out_specs=pl.BlockSpec((_R, N_CLS), lambda i: (i, 0)),
      out_shape=jax.ShapeDtypeStruct((N, N_CLS), jnp.float32),
  )(p, nd, b3p)


def kernel(x, edge_index, W1, b1, W2, b2, W3, b3):
  pad = jnp.full((E_PAD - E,), PAD_IDX, jnp.int32)
  src2 = jnp.concatenate([edge_index[0], pad]).reshape(IDX_ROWS, CHUNK)
  dst2 = jnp.concatenate([edge_index[1], pad]).reshape(IDX_ROWS, CHUNK)
  w3p = jnp.pad(W3, ((0, 0), (0, D_CLS_PAD - N_CLS)))
  b3p = jnp.pad(b3, (0, D_CLS_PAD - N_CLS)).reshape(1, -1)

  degp = _deg_call(src2, dst2)
  hs, nd, ns = _pre_call(degp, x)
  p1 = _agg(hs, src2, dst2)
  h2s = _mid_call(p1, nd, ns, W1, b1)
  p2 = _agg(h2s, src2, dst2)
  m3 = _mid3_call(p2, nd, ns, W2, b2, w3p)
  p3 = _agg(m3, src2, dst2)
  return _fin_call(p3, nd, b3p)


# speedup vs baseline: 1.3446x; 1.1256x over previous
import functools

import jax
import jax.numpy as jnp
from jax import lax
from jax.experimental import pallas as pl
from jax.experimental.pallas import tpu as pltpu
from jax.experimental.pallas import tpu_sc as plsc

N = 10000
N_PAD = 10240
E = 320000
D_IN = 128
D_HID = 128
N_CLS = 40
D_CLS_PAD = 48

NC = 2
NS = 16
ROWS_PER_SUB = N_PAD // NS
E_PER_CORE = E // NC
E_PER_TILE = E_PER_CORE // NS
CHUNK = 96
TILE_CHUNKS = 106
NGRP = 1
GRP = TILE_CHUNKS // NGRP
E_PAD = NC * NS * CHUNK * TILE_CHUNKS
PAD_IDX = N_PAD - 1
IDX_ROWS = E_PAD // CHUNK

_R = 1280
_GRID = N_PAD // _R


def _sc_mesh():
  return plsc.VectorSubcoreMesh(core_axis_name="c", subcore_axis_name="s")


def _deg_body(src2_hbm, dst2_hbm, zeros_hbm, ones_hbm, out_hbm,
              isrc_v, idst_v, ones_v, degd_sh, degs_sh):
  c = lax.axis_index("c")
  s = lax.axis_index("s")
  r0 = s * ROWS_PER_SUB
  rows = pl.ds(r0, ROWS_PER_SUB)
  pltpu.sync_copy(zeros_hbm, degd_sh.at[rows])
  pltpu.sync_copy(zeros_hbm, degs_sh.at[rows])
  pltpu.sync_copy(ones_hbm, ones_v)
  row0 = (c * NS + s) * TILE_CHUNKS
  pltpu.sync_copy(src2_hbm.at[pl.ds(row0, TILE_CHUNKS)], isrc_v)
  pltpu.sync_copy(dst2_hbm.at[pl.ds(row0, TILE_CHUNKS)], idst_v)
  plsc.subcore_barrier()

  def step(i, carry):
    pltpu.sync_copy(ones_v, degd_sh.at[idst_v.at[i]], add=True)
    pltpu.sync_copy(ones_v, degs_sh.at[isrc_v.at[i]], add=True)
    return carry

  lax.fori_loop(0, TILE_CHUNKS, step, 0)
  plsc.subcore_barrier()
  pltpu.sync_copy(degd_sh.at[rows], out_hbm.at[c, 0, rows])
  pltpu.sync_copy(degs_sh.at[rows], out_hbm.at[c, 1, rows])


def _deg_call(src2, dst2):
  zeros = jnp.zeros((ROWS_PER_SUB, 16), jnp.float32)
  ones = jnp.ones((CHUNK, 16), jnp.float32)
  fn = pl.kernel(
      _deg_body,
      out_type=jax.ShapeDtypeStruct((NC, 2, N_PAD, 16), jnp.float32),
      mesh=_sc_mesh(),
      compiler_params=pltpu.CompilerParams(use_tc_tiling_on_sc=False),
      scratch_types=[
          pltpu.VMEM((TILE_CHUNKS, CHUNK), jnp.int32),
          pltpu.VMEM((TILE_CHUNKS, CHUNK), jnp.int32),
          pltpu.VMEM((CHUNK, 16), jnp.float32),
          pltpu.VMEM_SHARED((N_PAD, 16), jnp.float32),
          pltpu.VMEM_SHARED((N_PAD, 16), jnp.float32),
      ],
  )
  return fn(src2, dst2, zeros, ones)


def _agg_body(h_hbm, src2_hbm, dst2_hbm, zeros_hbm, out_hbm,
              isrc_v, idst_v, rows_a, rows_b, acc_sh, sem_a, sem_b):
  c = lax.axis_index("c")
  s = lax.axis_index("s")
  r0 = s * ROWS_PER_SUB
  rows = pl.ds(r0, ROWS_PER_SUB)
  with jax.named_scope("agg_init"):
    pltpu.sync_copy(zeros_hbm, acc_sh.at[rows])
    tile_row0 = (c * NS + s) * TILE_CHUNKS
    plsc.subcore_barrier()

  for g in range(NGRP):
    row0 = tile_row0 + g * GRP
    pltpu.sync_copy(src2_hbm.at[pl.ds(row0, GRP)], isrc_v)
    pltpu.sync_copy(dst2_hbm.at[pl.ds(row0, GRP)], idst_v)
    pltpu.async_copy(h_hbm.at[isrc_v.at[0]], rows_a, sem_a)
    pltpu.async_copy(h_hbm.at[isrc_v.at[1]], rows_b, sem_b)

    def step(i, carry):
      ja = 2 * i
      jb = ja + 1
      pltpu.make_async_copy(h_hbm.at[isrc_v.at[ja]], rows_a, sem_a).wait()
      pltpu.sync_copy(rows_a, acc_sh.at[idst_v.at[ja]], add=True)
      pltpu.async_copy(h_hbm.at[isrc_v.at[ja + 2]], rows_a, sem_a)
      pltpu.make_async_copy(h_hbm.at[isrc_v.at[jb]], rows_b, sem_b).wait()
      pltpu.sync_copy(rows_b, acc_sh.at[idst_v.at[jb]], add=True)
      pltpu.async_copy(h_hbm.at[isrc_v.at[jb + 2]], rows_b, sem_b)
      return carry

    with jax.named_scope("agg_loop"):
      lax.fori_loop(0, GRP // 2 - 1, step, 0)
    last = GRP - 2
    pltpu.make_async_copy(h_hbm.at[isrc_v.at[last]], rows_a, sem_a).wait()
    pltpu.sync_copy(rows_a, acc_sh.at[idst_v.at[last]], add=True)
    pltpu.make_async_copy(h_hbm.at[isrc_v.at[last + 1]], rows_b, sem_b).wait()
    pltpu.sync_copy(rows_b, acc_sh.at[idst_v.at[last + 1]], add=True)

  with jax.named_scope("agg_wb"):
    plsc.subcore_barrier()
    pltpu.sync_copy(acc_sh.at[rows], out_hbm.at[c, rows])


def _make_agg(d):
  params = pltpu.CompilerParams(use_tc_tiling_on_sc=False)
  return pl.kernel(
      _agg_body,
      out_type=jax.ShapeDtypeStruct((NC, N_PAD, d), jnp.float32),
      mesh=_sc_mesh(),
      compiler_params=params,
      scratch_types=[
          pltpu.VMEM((GRP, CHUNK), jnp.int32),
          pltpu.VMEM((GRP, CHUNK), jnp.int32),
          pltpu.VMEM((CHUNK, d), jnp.float32),
          pltpu.VMEM((CHUNK, d), jnp.float32),
          pltpu.VMEM_SHARED((N_PAD, d), jnp.float32),
          pltpu.SemaphoreType.DMA,
          pltpu.SemaphoreType.DMA,
      ],
  )


def _agg(h, src2, dst2):
  d = h.shape[1]
  zeros = jnp.zeros((ROWS_PER_SUB, d), jnp.float32)
  return _make_agg(d)(h, src2, dst2, zeros)


def _row_keep():
  rid = pl.program_id(0) * _R + lax.broadcasted_iota(jnp.int32, (_R, 1), 0)
  return rid < N


def _pre_body(degp_ref, x_ref, hs_ref, nd_ref, ns_ref):
  dd = degp_ref[0, 0] + degp_ref[1, 0]
  dsr = degp_ref[0, 1] + degp_ref[1, 1]
  nd = lax.rsqrt(jnp.where(dd > 0.0, dd, 1.0))
  ns = lax.rsqrt(jnp.where(dsr > 0.0, dsr, 1.0))
  nd_ref[...] = nd
  ns_ref[...] = ns
  hs_ref[...] = jnp.where(_row_keep(), x_ref[...] * ns[:, 0:1], 0.0)


def _pre_call(degp, x):
  return pl.pallas_call(
      _pre_body,
      grid=(_GRID,),
      in_specs=[
          pl.BlockSpec((NC, 2, _R, 16), lambda i: (0, 0, i, 0)),
          pl.BlockSpec((_R, D_IN), lambda i: (i, 0)),
      ],
      out_specs=[
          pl.BlockSpec((_R, D_IN), lambda i: (i, 0)),
          pl.BlockSpec((_R, 16), lambda i: (i, 0)),
          pl.BlockSpec((_R, 16), lambda i: (i, 0)),
      ],
      out_shape=[
          jax.ShapeDtypeStruct((N_PAD, D_IN), jnp.float32),
          jax.ShapeDtypeStruct((N_PAD, 16), jnp.float32),
          jax.ShapeDtypeStruct((N_PAD, 16), jnp.float32),
      ],
  )(degp, x)


def _mid_body(p_ref, nd_ref, ns_ref, w_ref, b_ref, o_ref):
  agg = (p_ref[0] + p_ref[1]) * nd_ref[:, 0:1]
  z = jnp.dot(agg, w_ref[...], preferred_element_type=jnp.float32) + b_ref[...]
  o_ref[...] = jnp.where(_row_keep(), jnp.maximum(z, 0.0) * ns_ref[:, 0:1], 0.0)


def _mid_call(p, nd, ns, w, b):
  return pl.pallas_call(
      _mid_body,
      grid=(_GRID,),
      in_specs=[
          pl.BlockSpec((NC, _R, D_HID), lambda i: (0, i, 0)),
          pl.BlockSpec((_R, 16), lambda i: (i, 0)),
          pl.BlockSpec((_R, 16), lambda i: (i, 0)),
          pl.BlockSpec((D_HID, D_HID), lambda i: (0, 0)),
          pl.BlockSpec((1, D_HID), lambda i: (0, 0)),
      ],
      out_specs=pl.BlockSpec((_R, D_HID), lambda i: (i, 0)),
      out_shape=jax.ShapeDtypeStruct((N_PAD, D_HID), jnp.float32),
  )(p, nd, ns, w, b.reshape(1, -1))


def _mid3_body(p_ref, nd_ref, ns_ref, w_ref, b_ref, w3_ref, o_ref):
  agg = (p_ref[0] + p_ref[1]) * nd_ref[:, 0:1]
  z = jnp.dot(agg, w_ref[...], preferred_element_type=jnp.float32) + b_ref[...]
  h = jnp.maximum(z, 0.0) * ns_ref[:, 0:1]
  m = jnp.dot(h, w3_ref[...], preferred_element_type=jnp.float32)
  o_ref[...] = jnp.where(_row_keep(), m, 0.0)


def _mid3_call(p, nd, ns, w, b, w3p):
  return pl.pallas_call(
      _mid3_body,
      grid=(_GRID,),
      in_specs=[
          pl.BlockSpec((NC, _R, D_HID), lambda i: (0, i, 0)),
          pl.BlockSpec((_R, 16), lambda i: (i, 0)),
          pl.BlockSpec((_R, 16), lambda i: (i, 0)),
          pl.BlockSpec((D_HID, D_HID), lambda i: (0, 0)),
          pl.BlockSpec((1, D_HID), lambda i: (0, 0)),
          pl.BlockSpec((D_HID, D_CLS_PAD), lambda i: (0, 0)),
      ],
      out_specs=pl.BlockSpec((_R, D_CLS_PAD), lambda i: (i, 0)),
      out_shape=jax.ShapeDtypeStruct((N_PAD, D_CLS_PAD), jnp.float32),
  )(p, nd, ns, w, b.reshape(1, -1), w3p)


def _fin_body(p_ref, nd_ref, b_ref, o_ref):
  agg = (p_ref[0] + p_ref[1]) * nd_ref[:, 0:1]
  o_ref[...] = (agg + b_ref[...])[:, :N_CLS]


def _fin_call(p, nd, b3p):
  return pl.pallas_call(
      _fin_body,
      grid=(_GRID,),
      in_specs=[
          pl.BlockSpec((NC, _R, D_CLS_PAD), lambda i: (0, i, 0)),
          pl.BlockSpec((_R, 16), lambda i: (i, 0)),
          pl.BlockSpec((1, D_CLS_PAD), lambda i: (0, 0)),
      ],
      out_specs=pl.BlockSpec((_R, N_CLS), lambda i: (i, 0)),
      out_shape=jax.ShapeDtypeStruct((N, N_CLS), jnp.float32),
  )(p, nd, b3p)


def kernel(x, edge_index, W1, b1, W2, b2, W3, b3):
  npad = E_PAD - E
  pad = jnp.full((npad,), PAD_IDX, jnp.int32)
  spread = (jnp.arange(npad, dtype=jnp.int32) * 7919) % N
  src2 = jnp.concatenate([edge_index[0], pad]).reshape(IDX_ROWS, CHUNK)
  dst2a = jnp.concatenate([edge_index[1], spread]).reshape(IDX_ROWS, CHUNK)
  dst2d = jnp.concatenate([edge_index[1], pad]).reshape(IDX_ROWS, CHUNK)
  w3p = jnp.pad(W3, ((0, 0), (0, D_CLS_PAD - N_CLS)))
  b3p = jnp.pad(b3, (0, D_CLS_PAD - N_CLS)).reshape(1, -1)

  degp = _deg_call(src2, dst2d)
  hs, nd, ns = _pre_call(degp, x)
  p1 = _agg(hs, src2, dst2a)
  h2s = _mid_call(p1, nd, ns, W1, b1)
  p2 = _agg(h2s, src2, dst2a)
  m3 = _mid3_call(p2, nd, ns, W2, b2, w3p)
  p3 = _agg(m3, src2, dst2a)
  return _fin_call(p3, nd, b3p)


# speedup vs baseline: 2.7846x; 2.0710x over previous
import functools

import jax
import jax.numpy as jnp
from jax import lax
from jax.experimental import pallas as pl
from jax.experimental.pallas import tpu as pltpu
from jax.experimental.pallas import tpu_sc as plsc

N = 10000
N_PAD = 10240
E = 320000
D_IN = 128
D_HID = 128
N_CLS = 40
D_CLS_PAD = 48

NC = 2
NS = 16
ROWS_PER_SUB = N_PAD // NS
E_PER_CORE = E // NC
E_PER_TILE = E_PER_CORE // NS
CHUNK = 96
TILE_CHUNKS = 106
NGRP = 1
GRP = TILE_CHUNKS // NGRP
E_PAD = NC * NS * CHUNK * TILE_CHUNKS
PAD_IDX = N_PAD - 1
IDX_ROWS = E_PAD // CHUNK

_R = 1280
_GRID = N_PAD // _R


def _sc_mesh():
  return plsc.VectorSubcoreMesh(core_axis_name="c", subcore_axis_name="s")


def _deg_body(src2_hbm, dst2_hbm, zeros_hbm, ones_hbm, out_hbm,
              isrc_v, idst_v, ones_v, degd_sh, degs_sh):
  c = lax.axis_index("c")
  s = lax.axis_index("s")
  r0 = s * ROWS_PER_SUB
  rows = pl.ds(r0, ROWS_PER_SUB)
  pltpu.sync_copy(zeros_hbm, degd_sh.at[rows])
  pltpu.sync_copy(zeros_hbm, degs_sh.at[rows])
  pltpu.sync_copy(ones_hbm, ones_v)
  row0 = (c * NS + s) * TILE_CHUNKS
  pltpu.sync_copy(src2_hbm.at[pl.ds(row0, TILE_CHUNKS)], isrc_v)
  pltpu.sync_copy(dst2_hbm.at[pl.ds(row0, TILE_CHUNKS)], idst_v)
  plsc.subcore_barrier()

  def step(i, carry):
    pltpu.sync_copy(ones_v, degd_sh.at[idst_v.at[i]], add=True)
    pltpu.sync_copy(ones_v, degs_sh.at[isrc_v.at[i]], add=True)
    return carry

  lax.fori_loop(0, TILE_CHUNKS, step, 0)
  plsc.subcore_barrier()
  pltpu.sync_copy(degd_sh.at[rows], out_hbm.at[c, 0, rows])
  pltpu.sync_copy(degs_sh.at[rows], out_hbm.at[c, 1, rows])


def _deg_call(src2, dst2):
  zeros = jnp.zeros((ROWS_PER_SUB, 16), jnp.float32)
  ones = jnp.ones((CHUNK, 16), jnp.float32)
  fn = pl.kernel(
      _deg_body,
      out_type=jax.ShapeDtypeStruct((NC, 2, N_PAD, 16), jnp.float32),
      mesh=_sc_mesh(),
      compiler_params=pltpu.CompilerParams(use_tc_tiling_on_sc=False),
      scratch_types=[
          pltpu.VMEM((TILE_CHUNKS, CHUNK), jnp.int32),
          pltpu.VMEM((TILE_CHUNKS, CHUNK), jnp.int32),
          pltpu.VMEM((CHUNK, 16), jnp.float32),
          pltpu.VMEM_SHARED((N_PAD, 16), jnp.float32),
          pltpu.VMEM_SHARED((N_PAD, 16), jnp.float32),
      ],
  )
  return fn(src2, dst2, zeros, ones)


def _agg_body(h_hbm, src2_hbm, dst2_hbm, zeros_hbm, out_hbm,
              isrc_v, idst_v, rows_a, rows_b, acc_sh, sem_a, sem_b):
  c = lax.axis_index("c")
  s = lax.axis_index("s")
  r0 = s * ROWS_PER_SUB
  rows = pl.ds(r0, ROWS_PER_SUB)
  with jax.named_scope("agg_init"):
    pltpu.sync_copy(zeros_hbm, acc_sh.at[rows])
    tile_row0 = (c * NS + s) * TILE_CHUNKS
    plsc.subcore_barrier()

  for g in range(NGRP):
    row0 = tile_row0 + g * GRP
    pltpu.sync_copy(src2_hbm.at[pl.ds(row0, GRP)], isrc_v)
    pltpu.sync_copy(dst2_hbm.at[pl.ds(row0, GRP)], idst_v)
    pltpu.async_copy(h_hbm.at[isrc_v.at[0]], rows_a, sem_a)
    pltpu.async_copy(h_hbm.at[isrc_v.at[1]], rows_b, sem_b)

    def step(i, carry):
      ja = 2 * i
      jb = ja + 1
      pltpu.make_async_copy(h_hbm.at[isrc_v.at[ja]], rows_a, sem_a).wait()
      pltpu.sync_copy(rows_a, acc_sh.at[idst_v.at[ja]], add=True)
      pltpu.async_copy(h_hbm.at[isrc_v.at[ja + 2]], rows_a, sem_a)
      pltpu.make_async_copy(h_hbm.at[isrc_v.at[jb]], rows_b, sem_b).wait()
      pltpu.sync_copy(rows_b, acc_sh.at[idst_v.at[jb]], add=True)
      pltpu.async_copy(h_hbm.at[isrc_v.at[jb + 2]], rows_b, sem_b)
      return carry

    with jax.named_scope("agg_loop"):
      lax.fori_loop(0, GRP // 2 - 1, step, 0)
    last = GRP - 2
    pltpu.make_async_copy(h_hbm.at[isrc_v.at[last]], rows_a, sem_a).wait()
    pltpu.sync_copy(rows_a, acc_sh.at[idst_v.at[last]], add=True)
    pltpu.make_async_copy(h_hbm.at[isrc_v.at[last + 1]], rows_b, sem_b).wait()
    pltpu.sync_copy(rows_b, acc_sh.at[idst_v.at[last + 1]], add=True)

  with jax.named_scope("agg_wb"):
    plsc.subcore_barrier()
    pltpu.sync_copy(acc_sh.at[rows], out_hbm.at[c, rows])


def _make_agg(d):
  params = pltpu.CompilerParams(use_tc_tiling_on_sc=False)
  return pl.kernel(
      _agg_body,
      out_type=jax.ShapeDtypeStruct((NC, N_PAD, d), jnp.float32),
      mesh=_sc_mesh(),
      compiler_params=params,
      scratch_types=[
          pltpu.VMEM((GRP, CHUNK), jnp.int32),
          pltpu.VMEM((GRP, CHUNK), jnp.int32),
          pltpu.VMEM((CHUNK, d), jnp.float32),
          pltpu.VMEM((CHUNK, d), jnp.float32),
          pltpu.VMEM_SHARED((N_PAD, d), jnp.float32),
          pltpu.SemaphoreType.DMA,
          pltpu.SemaphoreType.DMA,
      ],
  )


def _agg(h, src2, dst2):
  d = h.shape[1]
  zeros = jnp.zeros((ROWS_PER_SUB, d), jnp.float32)
  return _make_agg(d)(h, src2, dst2, zeros)


def _row_keep():
  rid = pl.program_id(0) * _R + lax.broadcasted_iota(jnp.int32, (_R, 1), 0)
  return rid < N


def _pre_body(degp_ref, x_ref, hs_ref, nd_ref, ns_ref):
  dd = degp_ref[0, 0] + degp_ref[1, 0]
  dsr = degp_ref[0, 1] + degp_ref[1, 1]
  nd = lax.rsqrt(jnp.where(dd > 0.0, dd, 1.0))
  ns = lax.rsqrt(jnp.where(dsr > 0.0, dsr, 1.0))
  nd_ref[...] = nd
  ns_ref[...] = ns
  hs_ref[...] = jnp.where(_row_keep(), x_ref[...] * ns[:, 0:1], 0.0)


def _pre_call(degp, x):
  return pl.pallas_call(
      _pre_body,
      grid=(_GRID,),
      in_specs=[
          pl.BlockSpec((NC, 2, _R, 16), lambda i: (0, 0, i, 0)),
          pl.BlockSpec((_R, D_IN), lambda i: (i, 0)),
      ],
      out_specs=[
          pl.BlockSpec((_R, D_IN), lambda i: (i, 0)),
          pl.BlockSpec((_R, 16), lambda i: (i, 0)),
          pl.BlockSpec((_R, 16), lambda i: (i, 0)),
      ],
      out_shape=[
          jax.ShapeDtypeStruct((N_PAD, D_IN), jnp.float32),
          jax.ShapeDtypeStruct((N_PAD, 16), jnp.float32),
          jax.ShapeDtypeStruct((N_PAD, 16), jnp.float32),
      ],
  )(degp, x)


def _mid_body(p_ref, nd_ref, ns_ref, w_ref, b_ref, o_ref):
  agg = (p_ref[0] + p_ref[1]) * nd_ref[:, 0:1]
  z = jnp.dot(agg, w_ref[...], preferred_element_type=jnp.float32) + b_ref[...]
  o_ref[...] = jnp.where(_row_keep(), jnp.maximum(z, 0.0) * ns_ref[:, 0:1], 0.0)


def _mid_call(p, nd, ns, w, b):
  return pl.pallas_call(
      _mid_body,
      grid=(_GRID,),
      in_specs=[
          pl.BlockSpec((NC, _R, D_HID), lambda i: (0, i, 0)),
          pl.BlockSpec((_R, 16), lambda i: (i, 0)),
          pl.BlockSpec((_R, 16), lambda i: (i, 0)),
          pl.BlockSpec((D_HID, D_HID), lambda i: (0, 0)),
          pl.BlockSpec((1, D_HID), lambda i: (0, 0)),
      ],
      out_specs=pl.BlockSpec((_R, D_HID), lambda i: (i, 0)),
      out_shape=jax.ShapeDtypeStruct((N_PAD, D_HID), jnp.float32),
  )(p, nd, ns, w, b.reshape(1, -1))


def _mid3_body(p_ref, nd_ref, ns_ref, w_ref, b_ref, w3_ref, o_ref):
  agg = (p_ref[0] + p_ref[1]) * nd_ref[:, 0:1]
  z = jnp.dot(agg, w_ref[...], preferred_element_type=jnp.float32) + b_ref[...]
  h = jnp.maximum(z, 0.0) * ns_ref[:, 0:1]
  m = jnp.dot(h, w3_ref[...], preferred_element_type=jnp.float32)
  o_ref[...] = jnp.where(_row_keep(), m, 0.0)


def _mid3_call(p, nd, ns, w, b, w3p):
  return pl.pallas_call(
      _mid3_body,
      grid=(_GRID,),
      in_specs=[
          pl.BlockSpec((NC, _R, D_HID), lambda i: (0, i, 0)),
          pl.BlockSpec((_R, 16), lambda i: (i, 0)),
          pl.BlockSpec((_R, 16), lambda i: (i, 0)),
          pl.BlockSpec((D_HID, D_HID), lambda i: (0, 0)),
          pl.BlockSpec((1, D_HID), lambda i: (0, 0)),
          pl.BlockSpec((D_HID, D_CLS_PAD), lambda i: (0, 0)),
      ],
      out_specs=pl.BlockSpec((_R, D_CLS_PAD), lambda i: (i, 0)),
      out_shape=jax.ShapeDtypeStruct((N_PAD, D_CLS_PAD), jnp.float32),
  )(p, nd, ns, w, b.reshape(1, -1), w3p)


def _fin_body(p_ref, nd_ref, b_ref, o_ref):
  agg = (p_ref[0] + p_ref[1]) * nd_ref[:, 0:1]
  o_ref[...] = (agg + b_ref[...])[:, :N_CLS]


def _fin_call(p, nd, b3p):
  return pl.pallas_call(
      _fin_body,
      grid=(_GRID,),
      in_specs=[
          pl.BlockSpec((NC, _R, D_CLS_PAD), lambda i: (0, i, 0)),
          pl.BlockSpec((_R, 16), lambda i: (i, 0)),
          pl.BlockSpec((1, D_CLS_PAD), lambda i: (0, 0)),
      ],
      out_specs=pl.BlockSpec((_R, N_CLS), lambda i: (i, 0)),
      out_shape=jax.ShapeDtypeStruct((N, N_CLS), jnp.float32),
  )(p, nd, b3p)


def kernel(x, edge_index, W1, b1, W2, b2, W3, b3):
  npad = E_PAD - E
  pad = jnp.full((npad,), PAD_IDX, jnp.int32)
  spread = (jnp.arange(npad, dtype=jnp.int32) * 7919) % N
  zrows = N + jnp.arange(npad, dtype=jnp.int32) % (N_PAD - N)
  src2 = jnp.concatenate([edge_index[0], zrows]).reshape(IDX_ROWS, CHUNK)
  dst2a = jnp.concatenate([edge_index[1], spread]).reshape(IDX_ROWS, CHUNK)
  dst2d = jnp.concatenate([edge_index[1], pad]).reshape(IDX_ROWS, CHUNK)
  w3p = jnp.pad(W3, ((0, 0), (0, D_CLS_PAD - N_CLS)))
  b3p = jnp.pad(b3, (0, D_CLS_PAD - N_CLS)).reshape(1, -1)

  degp = _deg_call(src2, dst2d)
  hs, nd, ns = _pre_call(degp, x)
  p1 = _agg(hs, src2, dst2a)
  h2s = _mid_call(p1, nd, ns, W1, b1)
  p2 = _agg(h2s, src2, dst2a)
  m3 = _mid3_call(p2, nd, ns, W2, b2, w3p)
  p3 = _agg(m3, src2, dst2a)
  return _fin_call(p3, nd, b3p)


# speedup vs baseline: 2.8706x; 1.0309x over previous
import functools

import jax
import jax.numpy as jnp
from jax import lax
from jax.experimental import pallas as pl
from jax.experimental.pallas import tpu as pltpu
from jax.experimental.pallas import tpu_sc as plsc

N = 10000
N_PAD = 10240
E = 320000
D_IN = 128
D_HID = 128
N_CLS = 40
D_CLS_PAD = 48

NC = 2
NS = 16
ROWS_PER_SUB = N_PAD // NS
E_PER_CORE = E // NC
E_PER_TILE = E_PER_CORE // NS
CHUNK = 128
TILE_CHUNKS = 80
NGRP = 2
GRP = TILE_CHUNKS // NGRP
E_PAD = NC * NS * CHUNK * TILE_CHUNKS
PAD_IDX = N_PAD - 1
IDX_ROWS = E_PAD // CHUNK

_R = 1280
_GRID = N_PAD // _R


def _sc_mesh():
  return plsc.VectorSubcoreMesh(core_axis_name="c", subcore_axis_name="s")


def _deg_body(src2_hbm, dst2_hbm, zeros_hbm, ones_hbm, out_hbm,
              isrc_v, idst_v, ones_v, degd_sh, degs_sh):
  c = lax.axis_index("c")
  s = lax.axis_index("s")
  r0 = s * ROWS_PER_SUB
  rows = pl.ds(r0, ROWS_PER_SUB)
  pltpu.sync_copy(zeros_hbm, degd_sh.at[rows])
  pltpu.sync_copy(zeros_hbm, degs_sh.at[rows])
  pltpu.sync_copy(ones_hbm, ones_v)
  row0 = (c * NS + s) * TILE_CHUNKS
  pltpu.sync_copy(src2_hbm.at[pl.ds(row0, TILE_CHUNKS)], isrc_v)
  pltpu.sync_copy(dst2_hbm.at[pl.ds(row0, TILE_CHUNKS)], idst_v)
  plsc.subcore_barrier()

  def step(i, carry):
    pltpu.sync_copy(ones_v, degd_sh.at[idst_v.at[i]], add=True)
    pltpu.sync_copy(ones_v, degs_sh.at[isrc_v.at[i]], add=True)
    return carry

  lax.fori_loop(0, TILE_CHUNKS, step, 0)
  plsc.subcore_barrier()
  pltpu.sync_copy(degd_sh.at[rows], out_hbm.at[c, 0, rows])
  pltpu.sync_copy(degs_sh.at[rows], out_hbm.at[c, 1, rows])


def _deg_call(src2, dst2):
  zeros = jnp.zeros((ROWS_PER_SUB, 16), jnp.float32)
  ones = jnp.ones((CHUNK, 16), jnp.float32)
  fn = pl.kernel(
      _deg_body,
      out_type=jax.ShapeDtypeStruct((NC, 2, N_PAD, 16), jnp.float32),
      mesh=_sc_mesh(),
      compiler_params=pltpu.CompilerParams(use_tc_tiling_on_sc=False),
      scratch_types=[
          pltpu.VMEM((TILE_CHUNKS, CHUNK), jnp.int32),
          pltpu.VMEM((TILE_CHUNKS, CHUNK), jnp.int32),
          pltpu.VMEM((CHUNK, 16), jnp.float32),
          pltpu.VMEM_SHARED((N_PAD, 16), jnp.float32),
          pltpu.VMEM_SHARED((N_PAD, 16), jnp.float32),
      ],
  )
  return fn(src2, dst2, zeros, ones)


def _agg_body(h_hbm, src2_hbm, dst2_hbm, zeros_hbm, out_hbm,
              isrc_v, idst_v, rows_a, rows_b, acc_sh, sem_a, sem_b):
  c = lax.axis_index("c")
  s = lax.axis_index("s")
  r0 = s * ROWS_PER_SUB
  rows = pl.ds(r0, ROWS_PER_SUB)
  with jax.named_scope("agg_init"):
    pltpu.sync_copy(zeros_hbm, acc_sh.at[rows])
    tile_row0 = (c * NS + s) * TILE_CHUNKS
    plsc.subcore_barrier()

  for g in range(NGRP):
    row0 = tile_row0 + g * GRP
    pltpu.sync_copy(src2_hbm.at[pl.ds(row0, GRP)], isrc_v)
    pltpu.sync_copy(dst2_hbm.at[pl.ds(row0, GRP)], idst_v)
    pltpu.async_copy(h_hbm.at[isrc_v.at[0]], rows_a, sem_a)
    pltpu.async_copy(h_hbm.at[isrc_v.at[1]], rows_b, sem_b)

    def step(i, carry):
      ja = 2 * i
      jb = ja + 1
      pltpu.make_async_copy(h_hbm.at[isrc_v.at[ja]], rows_a, sem_a).wait()
      pltpu.sync_copy(rows_a, acc_sh.at[idst_v.at[ja]], add=True)
      pltpu.async_copy(h_hbm.at[isrc_v.at[ja + 2]], rows_a, sem_a)
      pltpu.make_async_copy(h_hbm.at[isrc_v.at[jb]], rows_b, sem_b).wait()
      pltpu.sync_copy(rows_b, acc_sh.at[idst_v.at[jb]], add=True)
      pltpu.async_copy(h_hbm.at[isrc_v.at[jb + 2]], rows_b, sem_b)
      return carry

    with jax.named_scope("agg_loop"):
      lax.fori_loop(0, GRP // 2 - 1, step, 0)
    last = GRP - 2
    pltpu.make_async_copy(h_hbm.at[isrc_v.at[last]], rows_a, sem_a).wait()
    pltpu.sync_copy(rows_a, acc_sh.at[idst_v.at[last]], add=True)
    pltpu.make_async_copy(h_hbm.at[isrc_v.at[last + 1]], rows_b, sem_b).wait()
    pltpu.sync_copy(rows_b, acc_sh.at[idst_v.at[last + 1]], add=True)

  with jax.named_scope("agg_wb"):
    plsc.subcore_barrier()
    pltpu.sync_copy(acc_sh.at[rows], out_hbm.at[c, rows])


def _make_agg(d):
  params = pltpu.CompilerParams(use_tc_tiling_on_sc=False)
  return pl.kernel(
      _agg_body,
      out_type=jax.ShapeDtypeStruct((NC, N_PAD, d), jnp.float32),
      mesh=_sc_mesh(),
      compiler_params=params,
      scratch_types=[
          pltpu.VMEM((GRP, CHUNK), jnp.int32),
          pltpu.VMEM((GRP, CHUNK), jnp.int32),
          pltpu.VMEM((CHUNK, d), jnp.float32),
          pltpu.VMEM((CHUNK, d), jnp.float32),
          pltpu.VMEM_SHARED((N_PAD, d), jnp.float32),
          pltpu.SemaphoreType.DMA,
          pltpu.SemaphoreType.DMA,
      ],
  )


def _agg(h, src2, dst2):
  d = h.shape[1]
  zeros = jnp.zeros((ROWS_PER_SUB, d), jnp.float32)
  return _make_agg(d)(h, src2, dst2, zeros)


def _row_keep():
  rid = pl.program_id(0) * _R + lax.broadcasted_iota(jnp.int32, (_R, 1), 0)
  return rid < N


def _pre_body(degp_ref, x_ref, hs_ref, nd_ref, ns_ref):
  dd = degp_ref[0, 0] + degp_ref[1, 0]
  dsr = degp_ref[0, 1] + degp_ref[1, 1]
  nd = lax.rsqrt(jnp.where(dd > 0.0, dd, 1.0))
  ns = lax.rsqrt(jnp.where(dsr > 0.0, dsr, 1.0))
  nd_ref[...] = nd
  ns_ref[...] = ns
  hs_ref[...] = jnp.where(_row_keep(), x_ref[...] * ns[:, 0:1], 0.0)


def _pre_call(degp, x):
  return pl.pallas_call(
      _pre_body,
      grid=(_GRID,),
      in_specs=[
          pl.BlockSpec((NC, 2, _R, 16), lambda i: (0, 0, i, 0)),
          pl.BlockSpec((_R, D_IN), lambda i: (i, 0)),
      ],
      out_specs=[
          pl.BlockSpec((_R, D_IN), lambda i: (i, 0)),
          pl.BlockSpec((_R, 16), lambda i: (i, 0)),
          pl.BlockSpec((_R, 16), lambda i: (i, 0)),
      ],
      out_shape=[
          jax.ShapeDtypeStruct((N_PAD, D_IN), jnp.float32),
          jax.ShapeDtypeStruct((N_PAD, 16), jnp.float32),
          jax.ShapeDtypeStruct((N_PAD, 16), jnp.float32),
      ],
  )(degp, x)


def _mid_body(p_ref, nd_ref, ns_ref, w_ref, b_ref, o_ref):
  agg = (p_ref[0] + p_ref[1]) * nd_ref[:, 0:1]
  z = jnp.dot(agg, w_ref[...], preferred_element_type=jnp.float32) + b_ref[...]
  o_ref[...] = jnp.where(_row_keep(), jnp.maximum(z, 0.0) * ns_ref[:, 0:1], 0.0)


def _mid_call(p, nd, ns, w, b):
  return pl.pallas_call(
      _mid_body,
      grid=(_GRID,),
      in_specs=[
          pl.BlockSpec((NC, _R, D_HID), lambda i: (0, i, 0)),
          pl.BlockSpec((_R, 16), lambda i: (i, 0)),
          pl.BlockSpec((_R, 16), lambda i: (i, 0)),
          pl.BlockSpec((D_HID, D_HID), lambda i: (0, 0)),
          pl.BlockSpec((1, D_HID), lambda i: (0, 0)),
      ],
      out_specs=pl.BlockSpec((_R, D_HID), lambda i: (i, 0)),
      out_shape=jax.ShapeDtypeStruct((N_PAD, D_HID), jnp.float32),
  )(p, nd, ns, w, b.reshape(1, -1))


def _mid3_body(p_ref, nd_ref, ns_ref, w_ref, b_ref, w3_ref, o_ref):
  agg = (p_ref[0] + p_ref[1]) * nd_ref[:, 0:1]
  z = jnp.dot(agg, w_ref[...], preferred_element_type=jnp.float32) + b_ref[...]
  h = jnp.maximum(z, 0.0) * ns_ref[:, 0:1]
  m = jnp.dot(h, w3_ref[...], preferred_element_type=jnp.float32)
  o_ref[...] = jnp.where(_row_keep(), m, 0.0)


def _mid3_call(p, nd, ns, w, b, w3p):
  return pl.pallas_call(
      _mid3_body,
      grid=(_GRID,),
      in_specs=[
          pl.BlockSpec((NC, _R, D_HID), lambda i: (0, i, 0)),
          pl.BlockSpec((_R, 16), lambda i: (i, 0)),
          pl.BlockSpec((_R, 16), lambda i: (i, 0)),
          pl.BlockSpec((D_HID, D_HID), lambda i: (0, 0)),
          pl.BlockSpec((1, D_HID), lambda i: (0, 0)),
          pl.BlockSpec((D_HID, D_CLS_PAD), lambda i: (0, 0)),
      ],
      out_specs=pl.BlockSpec((_R, D_CLS_PAD), lambda i: (i, 0)),
      out_shape=jax.ShapeDtypeStruct((N_PAD, D_CLS_PAD), jnp.float32),
  )(p, nd, ns, w, b.reshape(1, -1), w3p)


def _fin_body(p_ref, nd_ref, b_ref, o_ref):
  agg = (p_ref[0] + p_ref[1]) * nd_ref[:, 0:1]
  o_ref[...] = (agg + b_ref[...])[:, :N_CLS]


def _fin_call(p, nd, b3p):
  return pl.pallas_call(
      _fin_body,
      grid=(_GRID,),
      in_specs=[
          pl.BlockSpec((NC, _R, D_CLS_PAD), lambda i: (0, i, 0)),
          pl.BlockSpec((_R, 16), lambda i: (i, 0)),
          pl.BlockSpec((1, D_CLS_PAD), lambda i: (0, 0)),
      ],
      out_specs=pl.BlockSpec((_R, N_CLS), lambda i: (i, 0)),
      out_shape=jax.ShapeDtypeStruct((N, N_CLS), jnp.float32),
  )(p, nd, b3p)


def kernel(x, edge_index, W1, b1, W2, b2, W3, b3):
  npad = E_PAD - E
  pad = jnp.full((npad,), PAD_IDX, jnp.int32)
  spread = (jnp.arange(npad, dtype=jnp.int32) * 7919) % N
  zrows = N + jnp.arange(npad, dtype=jnp.int32) % (N_PAD - N)
  src2 = jnp.concatenate([edge_index[0], zrows]).reshape(IDX_ROWS, CHUNK)
  dst2a = jnp.concatenate([edge_index[1], spread]).reshape(IDX_ROWS, CHUNK)
  dst2d = jnp.concatenate([edge_index[1], pad]).reshape(IDX_ROWS, CHUNK)
  w3p = jnp.pad(W3, ((0, 0), (0, D_CLS_PAD - N_CLS)))
  b3p = jnp.pad(b3, (0, D_CLS_PAD - N_CLS)).reshape(1, -1)

  degp = _deg_call(src2, dst2d)
  hs, nd, ns = _pre_call(degp, x)
  p1 = _agg(hs, src2, dst2a)
  h2s = _mid_call(p1, nd, ns, W1, b1)
  p2 = _agg(h2s, src2, dst2a)
  m3 = _mid3_call(p2, nd, ns, W2, b2, w3p)
  p3 = _agg(m3, src2, dst2a)
  return _fin_call(p3, nd, b3p)


# speedup vs baseline: 2.9096x; 1.0136x over previous
import functools

import jax
import jax.numpy as jnp
from jax import lax
from jax.experimental import pallas as pl
from jax.experimental.pallas import tpu as pltpu
from jax.experimental.pallas import tpu_sc as plsc

N = 10000
N_PAD = 10240
E = 320000
D_IN = 128
D_HID = 128
N_CLS = 40
D_CLS_PAD = 48

NC = 2
NS = 16
ROWS_PER_SUB = N_PAD // NS
E_PER_CORE = E // NC
E_PER_TILE = E_PER_CORE // NS
CHUNK = 128
TILE_CHUNKS = 80
NGRP = 2
GRP = TILE_CHUNKS // NGRP
E_PAD = NC * NS * CHUNK * TILE_CHUNKS
PAD_IDX = N_PAD - 1
IDX_ROWS = E_PAD // CHUNK

_R = 2048
_GRID = N_PAD // _R


def _sc_mesh():
  return plsc.VectorSubcoreMesh(core_axis_name="c", subcore_axis_name="s")


def _deg_body(src2_hbm, dst2_hbm, zeros_hbm, ones_hbm, out_hbm,
              isrc_v, idst_v, ones_v, degd_sh, degs_sh):
  c = lax.axis_index("c")
  s = lax.axis_index("s")
  r0 = s * ROWS_PER_SUB
  rows = pl.ds(r0, ROWS_PER_SUB)
  pltpu.sync_copy(zeros_hbm, degd_sh.at[rows])
  pltpu.sync_copy(zeros_hbm, degs_sh.at[rows])
  pltpu.sync_copy(ones_hbm, ones_v)
  row0 = (c * NS + s) * TILE_CHUNKS
  pltpu.sync_copy(src2_hbm.at[pl.ds(row0, TILE_CHUNKS)], isrc_v)
  pltpu.sync_copy(dst2_hbm.at[pl.ds(row0, TILE_CHUNKS)], idst_v)
  plsc.subcore_barrier()

  def step(i, carry):
    pltpu.sync_copy(ones_v, degd_sh.at[idst_v.at[i]], add=True)
    pltpu.sync_copy(ones_v, degs_sh.at[isrc_v.at[i]], add=True)
    return carry

  lax.fori_loop(0, TILE_CHUNKS, step, 0)
  plsc.subcore_barrier()
  pltpu.sync_copy(degd_sh.at[rows], out_hbm.at[c, 0, rows])
  pltpu.sync_copy(degs_sh.at[rows], out_hbm.at[c, 1, rows])


def _deg_call(src2, dst2):
  zeros = jnp.zeros((ROWS_PER_SUB, 16), jnp.float32)
  ones = jnp.ones((CHUNK, 16), jnp.float32)
  fn = pl.kernel(
      _deg_body,
      out_type=jax.ShapeDtypeStruct((NC, 2, N_PAD, 16), jnp.float32),
      mesh=_sc_mesh(),
      compiler_params=pltpu.CompilerParams(use_tc_tiling_on_sc=False),
      scratch_types=[
          pltpu.VMEM((TILE_CHUNKS, CHUNK), jnp.int32),
          pltpu.VMEM((TILE_CHUNKS, CHUNK), jnp.int32),
          pltpu.VMEM((CHUNK, 16), jnp.float32),
          pltpu.VMEM_SHARED((N_PAD, 16), jnp.float32),
          pltpu.VMEM_SHARED((N_PAD, 16), jnp.float32),
      ],
  )
  return fn(src2, dst2, zeros, ones)


def _agg_body(h_hbm, src2_hbm, dst2_hbm, zeros_hbm, out_hbm,
              isrc_v, idst_v, rows_a, rows_b, acc_sh, sem_a, sem_b):
  c = lax.axis_index("c")
  s = lax.axis_index("s")
  r0 = s * ROWS_PER_SUB
  rows = pl.ds(r0, ROWS_PER_SUB)
  with jax.named_scope("agg_init"):
    pltpu.sync_copy(zeros_hbm, acc_sh.at[rows])
    tile_row0 = (c * NS + s) * TILE_CHUNKS
    plsc.subcore_barrier()

  for g in range(NGRP):
    row0 = tile_row0 + g * GRP
    pltpu.sync_copy(src2_hbm.at[pl.ds(row0, GRP)], isrc_v)
    pltpu.sync_copy(dst2_hbm.at[pl.ds(row0, GRP)], idst_v)
    pltpu.async_copy(h_hbm.at[isrc_v.at[0]], rows_a, sem_a)
    pltpu.async_copy(h_hbm.at[isrc_v.at[1]], rows_b, sem_b)

    def step(i, carry):
      ja = 2 * i
      jb = ja + 1
      pltpu.make_async_copy(h_hbm.at[isrc_v.at[ja]], rows_a, sem_a).wait()
      pltpu.sync_copy(rows_a, acc_sh.at[idst_v.at[ja]], add=True)
      pltpu.async_copy(h_hbm.at[isrc_v.at[ja + 2]], rows_a, sem_a)
      pltpu.make_async_copy(h_hbm.at[isrc_v.at[jb]], rows_b, sem_b).wait()
      pltpu.sync_copy(rows_b, acc_sh.at[idst_v.at[jb]], add=True)
      pltpu.async_copy(h_hbm.at[isrc_v.at[jb + 2]], rows_b, sem_b)
      return carry

    with jax.named_scope("agg_loop"):
      lax.fori_loop(0, GRP // 2 - 1, step, 0)
    last = GRP - 2
    pltpu.make_async_copy(h_hbm.at[isrc_v.at[last]], rows_a, sem_a).wait()
    pltpu.sync_copy(rows_a, acc_sh.at[idst_v.at[last]], add=True)
    pltpu.make_async_copy(h_hbm.at[isrc_v.at[last + 1]], rows_b, sem_b).wait()
    pltpu.sync_copy(rows_b, acc_sh.at[idst_v.at[last + 1]], add=True)

  with jax.named_scope("agg_wb"):
    plsc.subcore_barrier()
    pltpu.sync_copy(acc_sh.at[rows], out_hbm.at[c, rows])


def _make_agg(d):
  params = pltpu.CompilerParams(use_tc_tiling_on_sc=False)
  return pl.kernel(
      _agg_body,
      out_type=jax.ShapeDtypeStruct((NC, N_PAD, d), jnp.float32),
      mesh=_sc_mesh(),
      compiler_params=params,
      scratch_types=[
          pltpu.VMEM((GRP, CHUNK), jnp.int32),
          pltpu.VMEM((GRP, CHUNK), jnp.int32),
          pltpu.VMEM((CHUNK, d), jnp.float32),
          pltpu.VMEM((CHUNK, d), jnp.float32),
          pltpu.VMEM_SHARED((N_PAD, d), jnp.float32),
          pltpu.SemaphoreType.DMA,
          pltpu.SemaphoreType.DMA,
      ],
  )


def _agg(h, src2, dst2):
  d = h.shape[1]
  zeros = jnp.zeros((ROWS_PER_SUB, d), jnp.float32)
  return _make_agg(d)(h, src2, dst2, zeros)


def _row_keep():
  rid = pl.program_id(0) * _R + lax.broadcasted_iota(jnp.int32, (_R, 1), 0)
  return rid < N


def _pre_body(degp_ref, x_ref, hs_ref, nd_ref, ns_ref):
  dd = degp_ref[0, 0] + degp_ref[1, 0]
  dsr = degp_ref[0, 1] + degp_ref[1, 1]
  nd = lax.rsqrt(jnp.where(dd > 0.0, dd, 1.0))
  ns = lax.rsqrt(jnp.where(dsr > 0.0, dsr, 1.0))
  nd_ref[...] = nd
  ns_ref[...] = ns
  hs_ref[...] = jnp.where(_row_keep(), x_ref[...] * ns[:, 0:1], 0.0)


def _pre_call(degp, x):
  return pl.pallas_call(
      _pre_body,
      grid=(_GRID,),
      in_specs=[
          pl.BlockSpec((NC, 2, _R, 16), lambda i: (0, 0, i, 0)),
          pl.BlockSpec((_R, D_IN), lambda i: (i, 0)),
      ],
      out_specs=[
          pl.BlockSpec((_R, D_IN), lambda i: (i, 0)),
          pl.BlockSpec((_R, 16), lambda i: (i, 0)),
          pl.BlockSpec((_R, 16), lambda i: (i, 0)),
      ],
      out_shape=[
          jax.ShapeDtypeStruct((N_PAD, D_IN), jnp.float32),
          jax.ShapeDtypeStruct((N_PAD, 16), jnp.float32),
          jax.ShapeDtypeStruct((N_PAD, 16), jnp.float32),
      ],
  )(degp, x)


def _mid_body(p_ref, nd_ref, ns_ref, w_ref, b_ref, o_ref):
  agg = (p_ref[0] + p_ref[1]) * nd_ref[:, 0:1]
  z = jnp.dot(agg, w_ref[...], preferred_element_type=jnp.float32) + b_ref[...]
  o_ref[...] = jnp.where(_row_keep(), jnp.maximum(z, 0.0) * ns_ref[:, 0:1], 0.0)


def _mid_call(p, nd, ns, w, b):
  return pl.pallas_call(
      _mid_body,
      grid=(_GRID,),
      in_specs=[
          pl.BlockSpec((NC, _R, D_HID), lambda i: (0, i, 0)),
          pl.BlockSpec((_R, 16), lambda i: (i, 0)),
          pl.BlockSpec((_R, 16), lambda i: (i, 0)),
          pl.BlockSpec((D_HID, D_HID), lambda i: (0, 0)),
          pl.BlockSpec((1, D_HID), lambda i: (0, 0)),
      ],
      out_specs=pl.BlockSpec((_R, D_HID), lambda i: (i, 0)),
      out_shape=jax.ShapeDtypeStruct((N_PAD, D_HID), jnp.float32),
  )(p, nd, ns, w, b.reshape(1, -1))


def _mid3_body(p_ref, nd_ref, ns_ref, w_ref, b_ref, w3_ref, o_ref):
  agg = (p_ref[0] + p_ref[1]) * nd_ref[:, 0:1]
  z = jnp.dot(agg, w_ref[...], preferred_element_type=jnp.float32) + b_ref[...]
  h = jnp.maximum(z, 0.0) * ns_ref[:, 0:1]
  m = jnp.dot(h, w3_ref[...], preferred_element_type=jnp.float32)
  o_ref[...] = jnp.where(_row_keep(), m, 0.0)


def _mid3_call(p, nd, ns, w, b, w3p):
  return pl.pallas_call(
      _mid3_body,
      grid=(_GRID,),
      in_specs=[
          pl.BlockSpec((NC, _R, D_HID), lambda i: (0, i, 0)),
          pl.BlockSpec((_R, 16), lambda i: (i, 0)),
          pl.BlockSpec((_R, 16), lambda i: (i, 0)),
          pl.BlockSpec((D_HID, D_HID), lambda i: (0, 0)),
          pl.BlockSpec((1, D_HID), lambda i: (0, 0)),
          pl.BlockSpec((D_HID, D_CLS_PAD), lambda i: (0, 0)),
      ],
      out_specs=pl.BlockSpec((_R, D_CLS_PAD), lambda i: (i, 0)),
      out_shape=jax.ShapeDtypeStruct((N_PAD, D_CLS_PAD), jnp.float32),
  )(p, nd, ns, w, b.reshape(1, -1), w3p)


def _fin_body(p_ref, nd_ref, b_ref, o_ref):
  agg = (p_ref[0] + p_ref[1]) * nd_ref[:, 0:1]
  o_ref[...] = (agg + b_ref[...])[:, :N_CLS]


def _fin_call(p, nd, b3p):
  return pl.pallas_call(
      _fin_body,
      grid=(_GRID,),
      in_specs=[
          pl.BlockSpec((NC, _R, D_CLS_PAD), lambda i: (0, i, 0)),
          pl.BlockSpec((_R, 16), lambda i: (i, 0)),
          pl.BlockSpec((1, D_CLS_PAD), lambda i: (0, 0)),
      ],
      out_specs=pl.BlockSpec((_R, N_CLS), lambda i: (i, 0)),
      out_shape=jax.ShapeDtypeStruct((N, N_CLS), jnp.float32),
  )(p, nd, b3p)


def kernel(x, edge_index, W1, b1, W2, b2, W3, b3):
  npad = E_PAD - E
  pad = jnp.full((npad,), PAD_IDX, jnp.int32)
  spread = (jnp.arange(npad, dtype=jnp.int32) * 7919) % N
  zrows = N + jnp.arange(npad, dtype=jnp.int32) % (N_PAD - N)
  src2 = jnp.concatenate([edge_index[0], zrows]).reshape(IDX_ROWS, CHUNK)
  dst2a = jnp.concatenate([edge_index[1], spread]).reshape(IDX_ROWS, CHUNK)
  dst2d = jnp.concatenate([edge_index[1], pad]).reshape(IDX_ROWS, CHUNK)
  w3p = jnp.pad(W3, ((0, 0), (0, D_CLS_PAD - N_CLS)))
  b3p = jnp.pad(b3, (0, D_CLS_PAD - N_CLS)).reshape(1, -1)

  degp = _deg_call(src2, dst2d)
  hs, nd, ns = _pre_call(degp, x)
  p1 = _agg(hs, src2, dst2a)
  h2s = _mid_call(p1, nd, ns, W1, b1)
  p2 = _agg(h2s, src2, dst2a)
  m3 = _mid3_call(p2, nd, ns, W2, b2, w3p)
  p3 = _agg(m3, src2, dst2a)
  return _fin_call(p3, nd, b3p)
